# Initial kernel scaffold; baseline (speedup 1.0000x reference)
import jax
import jax.numpy as jnp
from jax.experimental import pallas as pl


def kernel(sequence, sequence_lenghts, pos_table):
    raise NotImplementedError("write your pallas kernel here")


# speedup vs baseline: 1.7751x; 1.7751x over previous
import jax
import jax.numpy as jnp
from jax.experimental import pallas as pl
from jax.experimental.pallas import tpu as pltpu

_BL = 256


def _body(seq_ref, tab_ref, lens_ref, out_ref):
    l = pl.program_id(0)
    b = pl.program_id(1)
    length = lens_ref[b]
    rowpos = jax.lax.broadcasted_iota(jnp.int32, (_BL, 1), 0) + l * _BL
    pe = jnp.where(rowpos < length, tab_ref[...], 0.0)
    out_ref[...] = seq_ref[...] + pe[None]


def kernel(sequence, sequence_lenghts, pos_table):
    B, L, D = sequence.shape
    lens = sequence_lenghts.astype(jnp.int32)
    tab = pos_table[1:]
    return pl.pallas_call(
        _body,
        grid=(L // _BL, B),
        in_specs=[
            pl.BlockSpec((1, _BL, D), lambda l, b: (b, l, 0)),
            pl.BlockSpec((_BL, D), lambda l, b: (l, 0)),
            pl.BlockSpec(memory_space=pltpu.SMEM),
        ],
        out_specs=pl.BlockSpec((1, _BL, D), lambda l, b: (b, l, 0)),
        out_shape=jax.ShapeDtypeStruct((B, L, D), jnp.float32),
    )(sequence, tab, lens)


# speedup vs baseline: 2.2086x; 1.2442x over previous
import jax
import jax.numpy as jnp
from jax.experimental import pallas as pl
from jax.experimental.pallas import tpu as pltpu

_BL = 512


def _body(seq_ref, tab_ref, lens_ref, out_ref):
    l = pl.program_id(0)
    b = pl.program_id(1)
    length = lens_ref[b]
    rowpos = jax.lax.broadcasted_iota(jnp.int32, (_BL, 1), 0) + l * _BL
    pe = jnp.where(rowpos < length, tab_ref[...], 0.0)
    out_ref[...] = seq_ref[...] + pe[None]


def kernel(sequence, sequence_lenghts, pos_table):
    B, L, D = sequence.shape
    lens = sequence_lenghts.astype(jnp.int32)
    tab = pos_table[1:]
    return pl.pallas_call(
        _body,
        grid=(L // _BL, B),
        in_specs=[
            pl.BlockSpec((1, _BL, D), lambda l, b: (b, l, 0)),
            pl.BlockSpec((_BL, D), lambda l, b: (l, 0)),
            pl.BlockSpec(memory_space=pltpu.SMEM),
        ],
        out_specs=pl.BlockSpec((1, _BL, D), lambda l, b: (b, l, 0)),
        out_shape=jax.ShapeDtypeStruct((B, L, D), jnp.float32),
    )(sequence, tab, lens)


# speedup vs baseline: 2.3565x; 1.0670x over previous
import jax
import jax.numpy as jnp
from jax.experimental import pallas as pl
from jax.experimental.pallas import tpu as pltpu

_BL = 1024


def _body(seq_ref, tab_ref, lens_ref, out_ref):
    l = pl.program_id(0)
    b = pl.program_id(1)
    length = lens_ref[b]
    rowpos = jax.lax.broadcasted_iota(jnp.int32, (_BL, 1), 0) + l * _BL
    pe = jnp.where(rowpos < length, tab_ref[...], 0.0)
    out_ref[...] = seq_ref[...] + pe[None]


def kernel(sequence, sequence_lenghts, pos_table):
    B, L, D = sequence.shape
    lens = sequence_lenghts.astype(jnp.int32)
    tab = pos_table[1:]
    return pl.pallas_call(
        _body,
        grid=(L // _BL, B),
        in_specs=[
            pl.BlockSpec((1, _BL, D), lambda l, b: (b, l, 0)),
            pl.BlockSpec((_BL, D), lambda l, b: (l, 0)),
            pl.BlockSpec(memory_space=pltpu.SMEM),
        ],
        out_specs=pl.BlockSpec((1, _BL, D), lambda l, b: (b, l, 0)),
        out_shape=jax.ShapeDtypeStruct((B, L, D), jnp.float32),
    )(sequence, tab, lens)


# speedup vs baseline: 2.5010x; 1.0613x over previous
import jax
import jax.numpy as jnp
from jax.experimental import pallas as pl
from jax.experimental.pallas import tpu as pltpu

_BL = 2048


def _body(seq_ref, tab_ref, lens_ref, out_ref):
    l = pl.program_id(0)
    b = pl.program_id(1)
    length = lens_ref[b]
    rowpos = jax.lax.broadcasted_iota(jnp.int32, (_BL, 1), 0) + l * _BL
    pe = jnp.where(rowpos < length, tab_ref[...], 0.0)
    out_ref[...] = seq_ref[...] + pe[None]


def kernel(sequence, sequence_lenghts, pos_table):
    B, L, D = sequence.shape
    lens = sequence_lenghts.astype(jnp.int32)
    tab = pos_table[1:]
    return pl.pallas_call(
        _body,
        grid=(L // _BL, B),
        in_specs=[
            pl.BlockSpec((1, _BL, D), lambda l, b: (b, l, 0)),
            pl.BlockSpec((_BL, D), lambda l, b: (l, 0)),
            pl.BlockSpec(memory_space=pltpu.SMEM),
        ],
        out_specs=pl.BlockSpec((1, _BL, D), lambda l, b: (b, l, 0)),
        out_shape=jax.ShapeDtypeStruct((B, L, D), jnp.float32),
    )(sequence, tab, lens)
